# Initial kernel scaffold; baseline (speedup 1.0000x reference)
import jax
import jax.numpy as jnp
from jax.experimental import pallas as pl


def kernel(x, table):
    raise NotImplementedError("write your pallas kernel here")


# speedup vs baseline: 1.8516x; 1.8516x over previous
import jax
import jax.numpy as jnp
from jax.experimental import pallas as pl

_BLK_S = 512


def _add_kernel(x_ref, t_ref, o_ref):
    o_ref[...] = x_ref[...] + t_ref[...]


def kernel(x, table):
    b, s, d = x.shape
    table_s = table[:s]
    grid = (s // _BLK_S, b)
    return pl.pallas_call(
        _add_kernel,
        grid=grid,
        in_specs=[
            pl.BlockSpec((1, _BLK_S, d), lambda j, i: (i, j, 0)),
            pl.BlockSpec((_BLK_S, d), lambda j, i: (j, 0)),
        ],
        out_specs=pl.BlockSpec((1, _BLK_S, d), lambda j, i: (i, j, 0)),
        out_shape=jax.ShapeDtypeStruct((b, s, d), x.dtype),
    )(x, table_s)


# speedup vs baseline: 1.9659x; 1.0617x over previous
import jax
import jax.numpy as jnp
from jax.experimental import pallas as pl

_BLK_S = 1024


def _add_kernel(x_ref, t_ref, o_ref):
    o_ref[...] = x_ref[...] + t_ref[...]


def kernel(x, table):
    b, s, d = x.shape
    table_s = table[:s]
    grid = (s // _BLK_S, b)
    return pl.pallas_call(
        _add_kernel,
        grid=grid,
        in_specs=[
            pl.BlockSpec((1, _BLK_S, d), lambda j, i: (i, j, 0)),
            pl.BlockSpec((_BLK_S, d), lambda j, i: (j, 0)),
        ],
        out_specs=pl.BlockSpec((1, _BLK_S, d), lambda j, i: (i, j, 0)),
        out_shape=jax.ShapeDtypeStruct((b, s, d), x.dtype),
    )(x, table_s)
